# Initial kernel scaffold; baseline (speedup 1.0000x reference)
import jax
import jax.numpy as jnp
from jax.experimental import pallas as pl


def kernel(pred_action_latents, embodiment_ids, W1, b1, W2, b2, action_mask):
    raise NotImplementedError("write your pallas kernel here")


# speedup vs baseline: 4.2839x; 4.2839x over previous
import jax
import jax.numpy as jnp
from jax.experimental import pallas as pl
from jax.experimental.pallas import tpu as pltpu

E = 8
D = 1024
H_DIM = 2048
MAX_A = 32


def _mlp_kernel(row_ids_ref, x_ref, w1_ref, b1_ref, w2_ref, b2_ref, mask_ref,
                out_ref):
    e = pl.program_id(0)
    x = x_ref[...]
    w1 = w1_ref[0]
    h = jnp.dot(x, w1, preferred_element_type=jnp.float32) + b1_ref[0]
    h = 0.5 * h * (1.0 + jax.lax.erf(h * 0.7071067811865476))
    dec = jnp.dot(h, w2_ref[0], preferred_element_type=jnp.float32)
    dec = (dec + b2_ref[0]) * mask_ref[0]
    sel = (row_ids_ref[:, 0] == e)[:, None]

    @pl.when(e == 0)
    def _():
        out_ref[...] = jnp.where(sel, dec, 0.0)

    @pl.when(e > 0)
    def _():
        out_ref[...] = jnp.where(sel, dec, out_ref[...])


def kernel(pred_action_latents, embodiment_ids, W1, b1, W2, b2, action_mask):
    B, T, _ = pred_action_latents.shape
    N = B * T
    x = pred_action_latents.reshape(N, D)
    row_ids = jnp.repeat(embodiment_ids.astype(jnp.int32), T)[:, None]

    out = pl.pallas_call(
        _mlp_kernel,
        grid=(E,),
        in_specs=[
            pl.BlockSpec((N, 1), lambda e: (0, 0)),
            pl.BlockSpec((N, D), lambda e: (0, 0)),
            pl.BlockSpec((1, D, H_DIM), lambda e: (e, 0, 0)),
            pl.BlockSpec((1, 1, H_DIM), lambda e: (e, 0, 0)),
            pl.BlockSpec((1, H_DIM, MAX_A), lambda e: (e, 0, 0)),
            pl.BlockSpec((1, 1, MAX_A), lambda e: (e, 0, 0)),
            pl.BlockSpec((1, 1, MAX_A), lambda e: (e, 0, 0)),
        ],
        out_specs=pl.BlockSpec((N, MAX_A), lambda e: (0, 0)),
        out_shape=jax.ShapeDtypeStruct((N, MAX_A), jnp.float32),
    )(row_ids, x, W1, b1[:, None, :], W2, b2[:, None, :],
      action_mask[:, None, :])
    return out.reshape(B, T, MAX_A)
